# baseline (speedup 1.0000x reference)
import functools

import jax
import jax.numpy as jnp
from jax import lax
from jax.experimental import pallas as pl
from jax.experimental.pallas import tpu as pltpu
from jax.experimental.pallas import tpu_sc as plsc

NC = 2
NS = 16
NW = NC * NS


def _layer_norm(h, g, beta):
    mu = jnp.mean(h, axis=-1, keepdims=True)
    var = jnp.mean(jnp.square(h - mu), axis=-1, keepdims=True)
    return (h - mu) * lax.rsqrt(var + 1e-5) * g + beta


def _preproj_body(nf_ref, ws_ref, wr_ref, nw1t_ref, nb1_ref,
                  ps_ref, pr_ref, nn1_ref):
    x = nf_ref[...]
    ps_ref[...] = jnp.dot(x, ws_ref[...], preferred_element_type=jnp.float32)
    pr_ref[...] = jnp.dot(x, wr_ref[...], preferred_element_type=jnp.float32)
    nn1_ref[...] = (jnp.dot(x, nw1t_ref[...], preferred_element_type=jnp.float32)
                    + nb1_ref[...])


def _preproj(nf, ws, wr, nw1t, nb1, block):
    n, d = nf.shape
    grid = n // block
    row_spec = pl.BlockSpec((block, d), lambda i: (i, 0))
    w_spec = pl.BlockSpec((d, d), lambda i: (0, 0))
    v_spec = pl.BlockSpec((1, d), lambda i: (0, 0))
    return pl.pallas_call(
        _preproj_body,
        grid=(grid,),
        in_specs=[row_spec, w_spec, w_spec, w_spec, v_spec],
        out_specs=[row_spec, row_spec, row_spec],
        out_shape=[jax.ShapeDtypeStruct((n, d), jnp.float32)] * 3,
    )(nf, ws, wr, nw1t, nb1)


def _make_gather(e, n, d, chunk, per):
    nch = per // chunk
    nsub = d // 16
    mesh = plsc.VectorSubcoreMesh(core_axis_name="c", subcore_axis_name="s", num_cores=NC, num_subcores=NS)

    @functools.partial(
        pl.kernel,
        out_type=jax.ShapeDtypeStruct((e, d), jnp.float32),
        mesh=mesh,
        scratch_types=[
            pltpu.VMEM((nch, chunk), jnp.int32),
            pltpu.VMEM((nch, chunk), jnp.int32),
            pltpu.VMEM((chunk, d), jnp.float32),
            pltpu.VMEM((chunk, d), jnp.float32),
            pltpu.VMEM((chunk, d), jnp.float32),
            pltpu.VMEM((chunk, d), jnp.float32),
            pltpu.VMEM((chunk, d), jnp.float32),
            pltpu.VMEM((chunk, d), jnp.float32),
            pltpu.VMEM((chunk, d), jnp.float32),
            pltpu.VMEM((chunk, d), jnp.float32),
            pltpu.SemaphoreType.DMA,
            pltpu.SemaphoreType.DMA,
            pltpu.SemaphoreType.DMA,
            pltpu.SemaphoreType.DMA,
            pltpu.SemaphoreType.DMA,
            pltpu.SemaphoreType.DMA,
            pltpu.SemaphoreType.DMA,
            pltpu.SemaphoreType.DMA,
        ],
    )
    def gather_k(ps_hbm, pr_hbm, s3_hbm, r3_hbm, g_hbm,
                 idx_s, idx_r, rs0, rr0, rs1, rr1, rs2, rr2, rs3, rr3,
                 sem_s0, sem_r0, sem_s1, sem_r1, sem_s2, sem_r2,
                 sem_s3, sem_r3):
        wid = lax.axis_index("s") * NC + lax.axis_index("c")
        cpa = pltpu.async_copy(s3_hbm.at[wid], idx_s, sem_s0)
        cpb = pltpu.async_copy(r3_hbm.at[wid], idx_r, sem_r0)
        cpa.wait()
        cpb.wait()

        bufs = ((rs0, rr0, sem_s0, sem_r0), (rs1, rr1, sem_s1, sem_r1),
                (rs2, rr2, sem_s2, sem_r2), (rs3, rr3, sem_s3, sem_r3))
        nbuf = len(bufs)

        def issue(i, b):
            rs, rr, ss, sr = bufs[b]
            pltpu.async_copy(ps_hbm.at[idx_s.at[i]], rs, ss)
            pltpu.async_copy(pr_hbm.at[idx_r.at[i]], rr, sr)

        for b in range(nbuf):
            issue(b, b)

        def outer(i0, carry):
            for b in range(nbuf):
                i = i0 * nbuf + b
                rs, rr, ss, sr = bufs[b]

                @pl.when(i < nch)
                def _():
                    pltpu.make_async_copy(ps_hbm.at[idx_s.at[i]], rs, ss).wait()
                    pltpu.make_async_copy(pr_hbm.at[idx_r.at[i]], rr, sr).wait()

                    @plsc.parallel_loop(0, chunk, step=1, unroll=2)
                    def add_row(r):
                        for j in range(nsub):
                            rs[r, pl.ds(j * 16, 16)] = (
                                rs[r, pl.ds(j * 16, 16)] + rr[r, pl.ds(j * 16, 16)])

                    base = wid * per + i * chunk
                    pltpu.sync_copy(rs, g_hbm.at[pl.ds(base, chunk)])

                    @pl.when(i + nbuf < nch)
                    def _issue_next():
                        issue(i + nbuf, b)

            return carry

        lax.fori_loop(0, (nch + nbuf - 1) // nbuf, outer, 0)

    return gather_k


def _edge_body(g_ref, ef_ref, w1e_ref, eb1_ref, w2_ref, eb2_ref,
               eg_ref, ebt_ref, ne0_ref, out_ref):
    ef = ef_ref[...]
    h = (g_ref[...]
         + jnp.dot(ef, w1e_ref[...], preferred_element_type=jnp.float32)
         + eb1_ref[...])
    h = jnp.maximum(h, 0.0)
    h = jnp.dot(h, w2_ref[...], preferred_element_type=jnp.float32) + eb2_ref[...]
    ne0 = _layer_norm(h, eg_ref[...], ebt_ref[...])
    ne0_ref[...] = ne0
    out_ref[...] = ne0 + ef


def _edge_mlp_part(g_part, ef, w1e, eb1, w2, eb2, eg, ebt, block, blk_off,
                   newe_prev=None):
    e, d = ef.shape
    ep = g_part.shape[0]
    grid = ep // block

    def _body(g_ref, ef_ref, w1e_ref, eb1_ref, w2_ref, eb2_ref,
              eg_ref, ebt_ref, prev_ref, ne0_ref, out_ref):
        del prev_ref
        _edge_body(g_ref, ef_ref, w1e_ref, eb1_ref, w2_ref, eb2_ref,
                   eg_ref, ebt_ref, ne0_ref, out_ref)

    part_spec = pl.BlockSpec((block, d), lambda i: (i, 0))
    full_spec = pl.BlockSpec((block, d), lambda i: (i + blk_off, 0))
    w_spec = pl.BlockSpec((d, d), lambda i: (0, 0))
    v_spec = pl.BlockSpec((1, d), lambda i: (0, 0))
    any_spec = pl.BlockSpec(memory_space=pl.ANY)
    if newe_prev is None:
        newe_prev = jnp.zeros((8, d), jnp.float32)
        aliases = {}
    else:
        aliases = {8: 1}
    return pl.pallas_call(
        _body,
        grid=(grid,),
        in_specs=[part_spec, full_spec, w_spec, v_spec, w_spec, v_spec,
                  v_spec, v_spec, any_spec],
        out_specs=[part_spec, full_spec],
        out_shape=[jax.ShapeDtypeStruct((ep, d), jnp.float32),
                   jax.ShapeDtypeStruct((e, d), jnp.float32)],
        input_output_aliases=aliases,
    )(g_part, ef, w1e, eb1, w2, eb2, eg, ebt, newe_prev)


def _make_scatter(e, n, d, chunk, per):
    nch = per // chunk
    slab = (n // NS) // 8 * 8
    rem = n - slab * NS
    mesh = plsc.VectorSubcoreMesh(core_axis_name="c", subcore_axis_name="s", num_cores=NC, num_subcores=NS)

    @functools.partial(
        pl.kernel,
        out_type=jax.ShapeDtypeStruct((NC, n, d), jnp.float32),
        mesh=mesh,
        scratch_types=[
            pltpu.VMEM((nch, chunk), jnp.int32),
            pltpu.VMEM((chunk, d), jnp.float32),
            pltpu.VMEM((chunk, d), jnp.float32),
            pltpu.VMEM((chunk, d), jnp.float32),
            pltpu.VMEM_SHARED((n, d), jnp.float32),
            pltpu.SemaphoreType.DMA,
            pltpu.SemaphoreType.DMA,
            pltpu.SemaphoreType.DMA,
            pltpu.SemaphoreType.DMA,
        ],
    )
    def scatter_k(ne_hbm, r3_hbm, z_hbm, out_hbm, idx_all, rows0, rows1,
                  rows2, acc_sh, sem_i, sem0, sem1, sem2):
        cid = lax.axis_index("c")
        sid = lax.axis_index("s")
        wid = cid * NS + sid
        cpi = pltpu.async_copy(r3_hbm.at[wid], idx_all, sem_i)

        pltpu.sync_copy(z_hbm.at[pl.ds(0, slab)], acc_sh.at[pl.ds(sid * slab, slab)])
        if rem:
            @pl.when(sid == NS - 1)
            def _():
                pltpu.sync_copy(z_hbm.at[pl.ds(0, rem)],
                                acc_sh.at[pl.ds(NS * slab, rem)])
        cpi.wait()

        bufs = ((rows0, sem0), (rows1, sem1), (rows2, sem2))
        nbuf = len(bufs)

        def issue(i, b):
            rows, sem = bufs[b]
            pltpu.async_copy(ne_hbm.at[pl.ds(wid * per + i * chunk, chunk)],
                             rows, sem)

        for b in range(nbuf):
            issue(b, b)
        plsc.subcore_barrier()

        def outer(i0, carry):
            for b in range(nbuf):
                i = i0 * nbuf + b
                rows, sem = bufs[b]

                @pl.when(i < nch)
                def _():
                    pltpu.make_async_copy(
                        ne_hbm.at[pl.ds(wid * per + i * chunk, chunk)],
                        rows, sem).wait()
                    pltpu.sync_copy(rows, acc_sh.at[idx_all.at[i]], add=True)

                    @pl.when(i + nbuf < nch)
                    def _issue_next():
                        issue(i + nbuf, b)

            return carry

        lax.fori_loop(0, (nch + nbuf - 1) // nbuf, outer, 0)
        plsc.subcore_barrier()
        pltpu.sync_copy(acc_sh.at[pl.ds(sid * slab, slab)],
                        out_hbm.at[cid, pl.ds(sid * slab, slab)])
        if rem:
            @pl.when(sid == NS - 1)
            def _():
                pltpu.sync_copy(acc_sh.at[pl.ds(NS * slab, rem)],
                                out_hbm.at[cid, pl.ds(NS * slab, rem)])

    return scatter_k


def _node_mlp(nf, nn1, parts_list, nw1b, nw2, nb2, ng, nbt, block):
    n, d = nf.shape
    grid = n // block
    np_ = len(parts_list)

    def _body(*refs):
        nf_ref, nn1_ref = refs[0], refs[1]
        parts_refs = refs[2:2 + np_]
        nw1b_ref, nw2_ref, nb2_ref, ng_ref, nbt_ref, out_ref = refs[2 + np_:]
        seg = parts_refs[0][0] + parts_refs[0][1]
        for p_ref in parts_refs[1:]:
            seg = seg + (p_ref[0] + p_ref[1])
        h = nn1_ref[...] + jnp.dot(seg, nw1b_ref[...],
                                   preferred_element_type=jnp.float32)
        h = jnp.maximum(h, 0.0)
        h = (jnp.dot(h, nw2_ref[...], preferred_element_type=jnp.float32)
             + nb2_ref[...])
        out_ref[...] = _layer_norm(h, ng_ref[...], nbt_ref[...]) + nf_ref[...]

    row_spec = pl.BlockSpec((block, d), lambda i: (i, 0))
    parts_spec = pl.BlockSpec((NC, block, d), lambda i: (0, i, 0))
    w_spec = pl.BlockSpec((d, d), lambda i: (0, 0))
    v_spec = pl.BlockSpec((1, d), lambda i: (0, 0))
    return pl.pallas_call(
        _body,
        grid=(grid,),
        in_specs=[row_spec, row_spec] + [parts_spec] * np_
                 + [w_spec, w_spec, v_spec, v_spec, v_spec],
        out_specs=row_spec,
        out_shape=jax.ShapeDtypeStruct((n, d), jnp.float32),
    )(nf, nn1, *parts_list, nw1b, nw2, nb2, ng, nbt)


def kernel(node_features, edge_features, senders, receivers,
           edge_w1, edge_b1, edge_w2, edge_b2, edge_g, edge_beta,
           node_w1, node_b1, node_w2, node_b2, node_g, node_beta):
    n, d = node_features.shape
    e = edge_features.shape[0]

    w1_s = edge_w1[:d]
    w1_r = edge_w1[d:2 * d]
    w1_e = edge_w1[2 * d:]
    nw1_t = node_w1[:d]
    nw1_b = node_w1[d:]

    eb1 = edge_b1.reshape(1, d)
    eb2 = edge_b2.reshape(1, d)
    eg = edge_g.reshape(1, d)
    ebt = edge_beta.reshape(1, d)
    nb1 = node_b1.reshape(1, d)
    nb2 = node_b2.reshape(1, d)
    ng = node_g.reshape(1, d)
    nbt = node_beta.reshape(1, d)

    ps, pr, nn1 = _preproj(node_features, w1_s, w1_r, nw1_t, nb1, block=2000)

    block = 8000
    chunk = 80
    sizes = [3 * e // 5, 2 * e // 5]
    zeros = jnp.zeros(((n // NS) // 8 * 8, d), dtype=jnp.float32)

    r_parts = []
    gparts = []
    off = 0
    for ep in sizes:
        per = ep // NW
        s3 = lax.slice_in_dim(senders, off, off + ep).reshape(
            NW, per // chunk, chunk)
        r3 = lax.slice_in_dim(receivers, off, off + ep).reshape(
            NW, per // chunk, chunk)
        r_parts.append(r3)
        gparts.append(_make_gather(ep, n, d, chunk=chunk, per=per)(
            ps, pr, s3, r3))
        off += ep

    parts_list = []
    newe = None
    off = 0
    for i, ep in enumerate(sizes):
        per = ep // NW
        ne0_i, newe = _edge_mlp_part(gparts[i], edge_features, w1_e, eb1,
                                     edge_w2, eb2, eg, ebt, block=block,
                                     blk_off=off // block,
                                     newe_prev=newe)
        parts_list.append(_make_scatter(ep, n, d, chunk=chunk, per=per)(
            ne0_i, r_parts[i], zeros))
        off += ep
    new_edge = newe

    new_node = _node_mlp(node_features, nn1, parts_list, nw1_b,
                         node_w2, nb2, ng, nbt, block=2000)
    return (new_node, new_edge)


# speedup vs baseline: 1.0041x; 1.0041x over previous
import functools

import jax
import jax.numpy as jnp
from jax import lax
from jax.experimental import pallas as pl
from jax.experimental.pallas import tpu as pltpu
from jax.experimental.pallas import tpu_sc as plsc

NC = 2
NS = 16
NW = NC * NS


def _layer_norm(h, g, beta):
    mu = jnp.mean(h, axis=-1, keepdims=True)
    var = jnp.mean(jnp.square(h - mu), axis=-1, keepdims=True)
    return (h - mu) * lax.rsqrt(var + 1e-5) * g + beta


def _preproj_body(nf_ref, ws_ref, wr_ref, nw1t_ref, nb1_ref,
                  ps_ref, pr_ref, nn1_ref):
    x = nf_ref[...]
    ps_ref[...] = jnp.dot(x, ws_ref[...], preferred_element_type=jnp.float32)
    pr_ref[...] = jnp.dot(x, wr_ref[...], preferred_element_type=jnp.float32)
    nn1_ref[...] = (jnp.dot(x, nw1t_ref[...], preferred_element_type=jnp.float32)
                    + nb1_ref[...])


def _preproj(nf, ws, wr, nw1t, nb1, block):
    n, d = nf.shape
    grid = n // block
    row_spec = pl.BlockSpec((block, d), lambda i: (i, 0))
    w_spec = pl.BlockSpec((d, d), lambda i: (0, 0))
    v_spec = pl.BlockSpec((1, d), lambda i: (0, 0))
    return pl.pallas_call(
        _preproj_body,
        grid=(grid,),
        in_specs=[row_spec, w_spec, w_spec, w_spec, v_spec],
        out_specs=[row_spec, row_spec, row_spec],
        out_shape=[jax.ShapeDtypeStruct((n, d), jnp.float32)] * 3,
    )(nf, ws, wr, nw1t, nb1)


def _make_gather(e, n, d, chunk, per):
    nch = per // chunk
    nsub = d // 16
    mesh = plsc.VectorSubcoreMesh(core_axis_name="c", subcore_axis_name="s", num_cores=NC, num_subcores=NS)

    @functools.partial(
        pl.kernel,
        out_type=jax.ShapeDtypeStruct((e, d), jnp.float32),
        mesh=mesh,
        scratch_types=[
            pltpu.VMEM((nch, chunk), jnp.int32),
            pltpu.VMEM((nch, chunk), jnp.int32),
            pltpu.VMEM((chunk, d), jnp.float32),
            pltpu.VMEM((chunk, d), jnp.float32),
            pltpu.VMEM((chunk, d), jnp.float32),
            pltpu.VMEM((chunk, d), jnp.float32),
            pltpu.VMEM((chunk, d), jnp.float32),
            pltpu.VMEM((chunk, d), jnp.float32),
            pltpu.SemaphoreType.DMA,
            pltpu.SemaphoreType.DMA,
            pltpu.SemaphoreType.DMA,
            pltpu.SemaphoreType.DMA,
            pltpu.SemaphoreType.DMA,
            pltpu.SemaphoreType.DMA,
        ],
    )
    def gather_k(ps_hbm, pr_hbm, s3_hbm, r3_hbm, g_hbm,
                 idx_s, idx_r, rs0, rr0, rs1, rr1, rs2, rr2,
                 sem_s0, sem_r0, sem_s1, sem_r1, sem_s2, sem_r2):
        wid = lax.axis_index("s") * NC + lax.axis_index("c")
        cpa = pltpu.async_copy(s3_hbm.at[wid], idx_s, sem_s0)
        cpb = pltpu.async_copy(r3_hbm.at[wid], idx_r, sem_r0)
        cpa.wait()
        cpb.wait()

        bufs = ((rs0, rr0, sem_s0, sem_r0), (rs1, rr1, sem_s1, sem_r1),
                (rs2, rr2, sem_s2, sem_r2))
        nbuf = len(bufs)

        def issue(i, b):
            rs, rr, ss, sr = bufs[b]
            pltpu.async_copy(ps_hbm.at[idx_s.at[i]], rs, ss)
            pltpu.async_copy(pr_hbm.at[idx_r.at[i]], rr, sr)

        for b in range(nbuf):
            issue(b, b)

        def outer(i0, carry):
            for b in range(nbuf):
                i = i0 * nbuf + b
                rs, rr, ss, sr = bufs[b]

                @pl.when(i < nch)
                def _():
                    pltpu.make_async_copy(ps_hbm.at[idx_s.at[i]], rs, ss).wait()
                    pltpu.make_async_copy(pr_hbm.at[idx_r.at[i]], rr, sr).wait()

                    @plsc.parallel_loop(0, chunk, step=1, unroll=2)
                    def add_row(r):
                        for j in range(nsub):
                            rs[r, pl.ds(j * 16, 16)] = (
                                rs[r, pl.ds(j * 16, 16)] + rr[r, pl.ds(j * 16, 16)])

                    base = wid * per + i * chunk
                    pltpu.sync_copy(rs, g_hbm.at[pl.ds(base, chunk)])

                    @pl.when(i + nbuf < nch)
                    def _issue_next():
                        issue(i + nbuf, b)

            return carry

        lax.fori_loop(0, (nch + nbuf - 1) // nbuf, outer, 0)

    return gather_k


def _edge_body(g_ref, ef_ref, w1e_ref, eb1_ref, w2_ref, eb2_ref,
               eg_ref, ebt_ref, ne0_ref, out_ref):
    ef = ef_ref[...]
    h = (g_ref[...]
         + jnp.dot(ef, w1e_ref[...], preferred_element_type=jnp.float32)
         + eb1_ref[...])
    h = jnp.maximum(h, 0.0)
    h = jnp.dot(h, w2_ref[...], preferred_element_type=jnp.float32) + eb2_ref[...]
    ne0 = _layer_norm(h, eg_ref[...], ebt_ref[...])
    ne0_ref[...] = ne0
    out_ref[...] = ne0 + ef


def _edge_mlp_part(g_part, ef, w1e, eb1, w2, eb2, eg, ebt, block, blk_off,
                   newe_prev=None):
    e, d = ef.shape
    ep = g_part.shape[0]
    grid = ep // block

    def _body(g_ref, ef_ref, w1e_ref, eb1_ref, w2_ref, eb2_ref,
              eg_ref, ebt_ref, prev_ref, ne0_ref, out_ref):
        del prev_ref
        _edge_body(g_ref, ef_ref, w1e_ref, eb1_ref, w2_ref, eb2_ref,
                   eg_ref, ebt_ref, ne0_ref, out_ref)

    part_spec = pl.BlockSpec((block, d), lambda i: (i, 0))
    full_spec = pl.BlockSpec((block, d), lambda i: (i + blk_off, 0))
    w_spec = pl.BlockSpec((d, d), lambda i: (0, 0))
    v_spec = pl.BlockSpec((1, d), lambda i: (0, 0))
    any_spec = pl.BlockSpec(memory_space=pl.ANY)
    if newe_prev is None:
        newe_prev = jnp.zeros((8, d), jnp.float32)
        aliases = {}
    else:
        aliases = {8: 1}
    return pl.pallas_call(
        _body,
        grid=(grid,),
        in_specs=[part_spec, full_spec, w_spec, v_spec, w_spec, v_spec,
                  v_spec, v_spec, any_spec],
        out_specs=[part_spec, full_spec],
        out_shape=[jax.ShapeDtypeStruct((ep, d), jnp.float32),
                   jax.ShapeDtypeStruct((e, d), jnp.float32)],
        input_output_aliases=aliases,
    )(g_part, ef, w1e, eb1, w2, eb2, eg, ebt, newe_prev)


def _make_scatter(e, n, d, chunk, per):
    nch = per // chunk
    slab = (n // NS) // 8 * 8
    rem = n - slab * NS
    mesh = plsc.VectorSubcoreMesh(core_axis_name="c", subcore_axis_name="s", num_cores=NC, num_subcores=NS)

    @functools.partial(
        pl.kernel,
        out_type=jax.ShapeDtypeStruct((NC, n, d), jnp.float32),
        mesh=mesh,
        scratch_types=[
            pltpu.VMEM((nch, chunk), jnp.int32),
            pltpu.VMEM((chunk, d), jnp.float32),
            pltpu.VMEM((chunk, d), jnp.float32),
            pltpu.VMEM((chunk, d), jnp.float32),
            pltpu.VMEM_SHARED((n, d), jnp.float32),
            pltpu.SemaphoreType.DMA,
            pltpu.SemaphoreType.DMA,
            pltpu.SemaphoreType.DMA,
            pltpu.SemaphoreType.DMA,
        ],
    )
    def scatter_k(ne_hbm, r3_hbm, z_hbm, out_hbm, idx_all, rows0, rows1,
                  rows2, acc_sh, sem_i, sem0, sem1, sem2):
        cid = lax.axis_index("c")
        sid = lax.axis_index("s")
        wid = cid * NS + sid
        cpi = pltpu.async_copy(r3_hbm.at[wid], idx_all, sem_i)

        pltpu.sync_copy(z_hbm.at[pl.ds(0, slab)], acc_sh.at[pl.ds(sid * slab, slab)])
        if rem:
            @pl.when(sid == NS - 1)
            def _():
                pltpu.sync_copy(z_hbm.at[pl.ds(0, rem)],
                                acc_sh.at[pl.ds(NS * slab, rem)])
        cpi.wait()

        bufs = ((rows0, sem0), (rows1, sem1), (rows2, sem2))
        nbuf = len(bufs)

        def issue(i, b):
            rows, sem = bufs[b]
            pltpu.async_copy(ne_hbm.at[pl.ds(wid * per + i * chunk, chunk)],
                             rows, sem)

        for b in range(nbuf):
            issue(b, b)
        plsc.subcore_barrier()

        def outer(i0, carry):
            for b in range(nbuf):
                i = i0 * nbuf + b
                rows, sem = bufs[b]

                @pl.when(i < nch)
                def _():
                    pltpu.make_async_copy(
                        ne_hbm.at[pl.ds(wid * per + i * chunk, chunk)],
                        rows, sem).wait()
                    pltpu.sync_copy(rows, acc_sh.at[idx_all.at[i]], add=True)

                    @pl.when(i + nbuf < nch)
                    def _issue_next():
                        issue(i + nbuf, b)

            return carry

        lax.fori_loop(0, (nch + nbuf - 1) // nbuf, outer, 0)
        plsc.subcore_barrier()
        pltpu.sync_copy(acc_sh.at[pl.ds(sid * slab, slab)],
                        out_hbm.at[cid, pl.ds(sid * slab, slab)])
        if rem:
            @pl.when(sid == NS - 1)
            def _():
                pltpu.sync_copy(acc_sh.at[pl.ds(NS * slab, rem)],
                                out_hbm.at[cid, pl.ds(NS * slab, rem)])

    return scatter_k


def _node_mlp(nf, nn1, parts_list, nw1b, nw2, nb2, ng, nbt, block):
    n, d = nf.shape
    grid = n // block
    np_ = len(parts_list)

    def _body(*refs):
        nf_ref, nn1_ref = refs[0], refs[1]
        parts_refs = refs[2:2 + np_]
        nw1b_ref, nw2_ref, nb2_ref, ng_ref, nbt_ref, out_ref = refs[2 + np_:]
        seg = parts_refs[0][0] + parts_refs[0][1]
        for p_ref in parts_refs[1:]:
            seg = seg + (p_ref[0] + p_ref[1])
        h = nn1_ref[...] + jnp.dot(seg, nw1b_ref[...],
                                   preferred_element_type=jnp.float32)
        h = jnp.maximum(h, 0.0)
        h = (jnp.dot(h, nw2_ref[...], preferred_element_type=jnp.float32)
             + nb2_ref[...])
        out_ref[...] = _layer_norm(h, ng_ref[...], nbt_ref[...]) + nf_ref[...]

    row_spec = pl.BlockSpec((block, d), lambda i: (i, 0))
    parts_spec = pl.BlockSpec((NC, block, d), lambda i: (0, i, 0))
    w_spec = pl.BlockSpec((d, d), lambda i: (0, 0))
    v_spec = pl.BlockSpec((1, d), lambda i: (0, 0))
    return pl.pallas_call(
        _body,
        grid=(grid,),
        in_specs=[row_spec, row_spec] + [parts_spec] * np_
                 + [w_spec, w_spec, v_spec, v_spec, v_spec],
        out_specs=row_spec,
        out_shape=jax.ShapeDtypeStruct((n, d), jnp.float32),
    )(nf, nn1, *parts_list, nw1b, nw2, nb2, ng, nbt)


def kernel(node_features, edge_features, senders, receivers,
           edge_w1, edge_b1, edge_w2, edge_b2, edge_g, edge_beta,
           node_w1, node_b1, node_w2, node_b2, node_g, node_beta):
    n, d = node_features.shape
    e = edge_features.shape[0]

    w1_s = edge_w1[:d]
    w1_r = edge_w1[d:2 * d]
    w1_e = edge_w1[2 * d:]
    nw1_t = node_w1[:d]
    nw1_b = node_w1[d:]

    eb1 = edge_b1.reshape(1, d)
    eb2 = edge_b2.reshape(1, d)
    eg = edge_g.reshape(1, d)
    ebt = edge_beta.reshape(1, d)
    nb1 = node_b1.reshape(1, d)
    nb2 = node_b2.reshape(1, d)
    ng = node_g.reshape(1, d)
    nbt = node_beta.reshape(1, d)

    ps, pr, nn1 = _preproj(node_features, w1_s, w1_r, nw1_t, nb1, block=2000)

    block = 8000
    chunk = 80
    sizes = [3 * e // 5, 2 * e // 5]
    zeros = jnp.zeros(((n // NS) // 8 * 8, d), dtype=jnp.float32)

    r_parts = []
    gparts = []
    off = 0
    for ep in sizes:
        per = ep // NW
        s3 = lax.slice_in_dim(senders, off, off + ep).reshape(
            NW, per // chunk, chunk)
        r3 = lax.slice_in_dim(receivers, off, off + ep).reshape(
            NW, per // chunk, chunk)
        r_parts.append(r3)
        gparts.append(_make_gather(ep, n, d, chunk=chunk, per=per)(
            ps, pr, s3, r3))
        off += ep

    parts_list = []
    newe = None
    off = 0
    for i, ep in enumerate(sizes):
        per = ep // NW
        ne0_i, newe = _edge_mlp_part(gparts[i], edge_features, w1_e, eb1,
                                     edge_w2, eb2, eg, ebt, block=block,
                                     blk_off=off // block,
                                     newe_prev=newe)
        parts_list.append(_make_scatter(ep, n, d, chunk=chunk, per=per)(
            ne0_i, r_parts[i], zeros))
        off += ep
    new_edge = newe

    new_node = _node_mlp(node_features, nn1, parts_list, nw1_b,
                         node_w2, nb2, ng, nbt, block=2000)
    return (new_node, new_edge)
